# Initial kernel scaffold; baseline (speedup 1.0000x reference)
import functools

import jax
import jax.numpy as jnp
from jax import lax
from jax.experimental import pallas as pl
from jax.experimental.pallas import tpu as pltpu
from jax.experimental.pallas import tpu_sc as plsc


def _mlp_body(e_ref, w1_ref, b1_ref, w2_ref, b2_ref, o_ref):
    h = jnp.dot(e_ref[...], w1_ref[...], preferred_element_type=jnp.float32)
    h = jnp.maximum(h + b1_ref[...], 0.0)
    o_ref[...] = (
        jnp.dot(h, w2_ref[...], preferred_element_type=jnp.float32) + b2_ref[...]
    )


def _table_mlp(esm, W1, b1, W2, b2, row_block):
    g, d = esm.shape
    hid = W1.shape[1]
    grid = (g + row_block - 1) // row_block
    return pl.pallas_call(
        _mlp_body,
        grid=(grid,),
        in_specs=[
            pl.BlockSpec((row_block, d), lambda i: (i, 0)),
            pl.BlockSpec((d, hid), lambda i: (0, 0)),
            pl.BlockSpec((1, hid), lambda i: (0, 0)),
            pl.BlockSpec((hid, hid), lambda i: (0, 0)),
            pl.BlockSpec((1, hid), lambda i: (0, 0)),
        ],
        out_specs=pl.BlockSpec((row_block, hid), lambda i: (i, 0)),
        out_shape=jax.ShapeDtypeStruct((g, hid), jnp.float32),
    )(esm, W1, b1.reshape(1, hid), W2, b2.reshape(1, hid))


def _gather_body(n_chunks, ch, nc, idx_hbm, tbl_hbm, out_hbm,
                 idx_v, rows_v, sem):
    wid = lax.axis_index("s") * nc + lax.axis_index("c")
    base = wid * (n_chunks * ch)
    for i in range(n_chunks):
        off = base + i * ch
        pltpu.sync_copy(idx_hbm.at[pl.ds(off, ch)], idx_v)
        pltpu.async_copy(tbl_hbm.at[idx_v], rows_v, sem).wait()
        pltpu.sync_copy(rows_v, out_hbm.at[pl.ds(off, ch)])


def _sc_gather(table, idx_flat):
    _, hid = table.shape
    bs = idx_flat.shape[0]
    info = plsc.get_sparse_core_info()
    nc, ns = info.num_cores, info.num_subcores
    nw = nc * ns
    per_w = bs // nw
    ch = 640
    n_chunks = per_w // ch
    body = functools.partial(_gather_body, n_chunks, ch, nc)
    kern = pl.kernel(
        body,
        out_type=jax.ShapeDtypeStruct((bs, hid), jnp.float32),
        mesh=plsc.VectorSubcoreMesh(core_axis_name="c", subcore_axis_name="s"),
        scratch_types=[
            pltpu.VMEM((ch,), jnp.int32),
            pltpu.VMEM((ch, hid), jnp.float32),
            pltpu.SemaphoreType.DMA,
        ],
    )
    return kern(idx_flat, table)


def kernel(pert_esm_indices, esm_embeddings, W1, b1, W2, b2):
    idx = pert_esm_indices
    if idx.shape[-1] == 1:
        idx = jnp.squeeze(idx, axis=-1)
    b, s = idx.shape
    hid = W1.shape[1]
    table = _table_mlp(esm_embeddings, W1, b1, W2, b2, row_block=800)
    idx_flat = idx.reshape(-1).astype(jnp.int32)
    out = _sc_gather(table, idx_flat)
    return out.reshape(b, s, hid)


# speedup vs baseline: 15.9450x; 15.9450x over previous
import functools

import jax
import jax.numpy as jnp
from jax import lax
from jax.experimental import pallas as pl
from jax.experimental.pallas import tpu as pltpu
from jax.experimental.pallas import tpu_sc as plsc


def _mlp_body(e_ref, w1_ref, b1_ref, w2_ref, b2_ref, o_ref):
    h = jnp.dot(e_ref[...], w1_ref[...], preferred_element_type=jnp.float32)
    h = jnp.maximum(h + b1_ref[...], 0.0)
    o_ref[...] = (
        jnp.dot(h, w2_ref[...], preferred_element_type=jnp.float32) + b2_ref[...]
    )


def _table_mlp(esm, W1, b1, W2, b2, row_block):
    g, d = esm.shape
    hid = W1.shape[1]
    grid = (g + row_block - 1) // row_block
    return pl.pallas_call(
        _mlp_body,
        grid=(grid,),
        in_specs=[
            pl.BlockSpec((row_block, d), lambda i: (i, 0)),
            pl.BlockSpec((d, hid), lambda i: (0, 0)),
            pl.BlockSpec((1, hid), lambda i: (0, 0)),
            pl.BlockSpec((hid, hid), lambda i: (0, 0)),
            pl.BlockSpec((1, hid), lambda i: (0, 0)),
        ],
        out_specs=pl.BlockSpec((row_block, hid), lambda i: (i, 0)),
        out_shape=jax.ShapeDtypeStruct((g, hid), jnp.float32),
    )(esm, W1, b1.reshape(1, hid), W2, b2.reshape(1, hid))


def _gather_body(n_chunks, ch, nc, idx_hbm, tbl_hbm, out_hbm,
                 idx_v, rows_v, sem):
    wid = lax.axis_index("s") * nc + lax.axis_index("c")
    base = wid * (n_chunks * ch)
    for i in range(n_chunks):
        off = base + i * ch
        pltpu.sync_copy(idx_hbm.at[pl.ds(off, ch)], idx_v)
        pltpu.async_copy(tbl_hbm.at[idx_v], rows_v, sem).wait()
        pltpu.sync_copy(rows_v, out_hbm.at[pl.ds(off, ch)])


def _sc_gather(table, idx_flat):
    _, hid = table.shape
    bs = idx_flat.shape[0]
    info = plsc.get_sparse_core_info()
    nc, ns = info.num_cores, info.num_subcores
    nw = nc * ns
    per_w = bs // nw
    ch = 640
    n_chunks = per_w // ch
    body = functools.partial(_gather_body, n_chunks, ch, nc)
    kern = pl.kernel(
        body,
        out_type=jax.ShapeDtypeStruct((bs, hid), jnp.float32),
        mesh=plsc.VectorSubcoreMesh(core_axis_name="c", subcore_axis_name="s"),
        scratch_types=[
            pltpu.VMEM((ch,), jnp.int32),
            pltpu.VMEM((ch, hid), jnp.float32),
            pltpu.SemaphoreType.DMA,
        ],
        compiler_params=pltpu.CompilerParams(use_tc_tiling_on_sc=False),
    )
    return kern(idx_flat, table)


def kernel(pert_esm_indices, esm_embeddings, W1, b1, W2, b2):
    idx = pert_esm_indices
    if idx.shape[-1] == 1:
        idx = jnp.squeeze(idx, axis=-1)
    b, s = idx.shape
    hid = W1.shape[1]
    table = _table_mlp(esm_embeddings, W1, b1, W2, b2, row_block=800)
    idx_flat = idx.reshape(-1).astype(jnp.int32)
    out = _sc_gather(table, idx_flat)
    return out.reshape(b, s, hid)
